# Initial kernel scaffold; baseline (speedup 1.0000x reference)
import jax
import jax.numpy as jnp
from jax.experimental import pallas as pl


def kernel(day, time, location_x, location_y, day_table, time_table, locx_table, locy_table):
    raise NotImplementedError("write your pallas kernel here")


# speedup vs baseline: 1.8336x; 1.8336x over previous
import functools

import jax
import jax.numpy as jnp
from jax import lax
from jax.experimental import pallas as pl
from jax.experimental.pallas import tpu as pltpu
from jax.experimental.pallas import tpu_sc as plsc

B, L = 16384, 50
BL = B * L
D_DAY, D_TIME, D_LOC = 32, 32, 128
D_OUT = D_DAY + D_TIME + 2 * D_LOC

NUM_CORES = 2
NUM_SUBCORES = 16
NW = NUM_CORES * NUM_SUBCORES
ROWS_PER_W = BL // NW
CHUNK = 128
NCHUNK = ROWS_PER_W // CHUNK


def _sc_embed(day_i, time_i, locx_i, locy_i, day_table, time_table,
              locx_table, locy_table):
  mesh = plsc.VectorSubcoreMesh(core_axis_name="c", subcore_axis_name="s")

  @functools.partial(
      pl.kernel,
      mesh=mesh,
      compiler_params=pltpu.CompilerParams(use_tc_tiling_on_sc=False),
      out_type=jax.ShapeDtypeStruct((BL, D_OUT), jnp.float32),
      scratch_types=[
          pltpu.VMEM((CHUNK,), jnp.int32),
          pltpu.VMEM((CHUNK,), jnp.int32),
          pltpu.VMEM((CHUNK,), jnp.int32),
          pltpu.VMEM((CHUNK,), jnp.int32),
          pltpu.VMEM((CHUNK, D_DAY), jnp.float32),
          pltpu.VMEM((CHUNK, D_TIME), jnp.float32),
          pltpu.VMEM((CHUNK, D_LOC), jnp.float32),
          pltpu.VMEM((CHUNK, D_LOC), jnp.float32),
          pltpu.SemaphoreType.DMA,
          pltpu.SemaphoreType.DMA,
      ],
  )
  def k(day_h, time_h, locx_h, locy_h, dt_h, tt_h, xt_h, yt_h, out_h,
        di_v, ti_v, xi_v, yi_v, dr_v, tr_v, xr_v, yr_v, gsem, wsem):
    wid = lax.axis_index("s") * NUM_CORES + lax.axis_index("c")
    w_base = wid * ROWS_PER_W

    def body(i, carry):
      base = w_base + i * CHUNK
      pltpu.sync_copy(day_h.at[pl.ds(base, CHUNK)], di_v)
      pltpu.sync_copy(time_h.at[pl.ds(base, CHUNK)], ti_v)
      pltpu.sync_copy(locx_h.at[pl.ds(base, CHUNK)], xi_v)
      pltpu.sync_copy(locy_h.at[pl.ds(base, CHUNK)], yi_v)
      g1 = pltpu.async_copy(dt_h.at[di_v], dr_v, gsem)
      g2 = pltpu.async_copy(tt_h.at[ti_v], tr_v, gsem)
      g3 = pltpu.async_copy(xt_h.at[xi_v], xr_v, gsem)
      g4 = pltpu.async_copy(yt_h.at[yi_v], yr_v, gsem)
      g1.wait()
      g2.wait()
      g3.wait()
      g4.wait()
      w1 = pltpu.async_copy(
          dr_v, out_h.at[pl.ds(base, CHUNK), pl.ds(0, D_DAY)], wsem)
      w2 = pltpu.async_copy(
          tr_v, out_h.at[pl.ds(base, CHUNK), pl.ds(D_DAY, D_TIME)], wsem)
      w3 = pltpu.async_copy(
          xr_v, out_h.at[pl.ds(base, CHUNK), pl.ds(D_DAY + D_TIME, D_LOC)],
          wsem)
      w4 = pltpu.async_copy(
          yr_v, out_h.at[pl.ds(base, CHUNK), pl.ds(D_DAY + D_TIME + D_LOC,
                                                   D_LOC)], wsem)
      w1.wait()
      w2.wait()
      w3.wait()
      w4.wait()
      return carry

    lax.fori_loop(0, NCHUNK, body, 0)

  return k(day_i, time_i, locx_i, locy_i, day_table, time_table, locx_table,
           locy_table)


def kernel(day, time, location_x, location_y, day_table, time_table,
           locx_table, locy_table):
  day_i = day.reshape(BL).astype(jnp.int32)
  time_i = time.reshape(BL).astype(jnp.int32)
  locx_i = location_x.reshape(BL).astype(jnp.int32)
  locy_i = location_y.reshape(BL).astype(jnp.int32)
  out = _sc_embed(day_i, time_i, locx_i, locy_i, day_table, time_table,
                  locx_table, locy_table)
  return out.reshape(B, L, D_OUT)


# speedup vs baseline: 1.8505x; 1.0092x over previous
import functools

import jax
import jax.numpy as jnp
from jax import lax
from jax.experimental import pallas as pl
from jax.experimental.pallas import tpu as pltpu
from jax.experimental.pallas import tpu_sc as plsc

B, L = 16384, 50
BL = B * L
D_DAY, D_TIME, D_LOC = 32, 32, 128
D_OUT = D_DAY + D_TIME + 2 * D_LOC

NUM_CORES = 2
NUM_SUBCORES = 16
NW = NUM_CORES * NUM_SUBCORES
ROWS_PER_W = BL // NW
CHUNK = 128
NCHUNK = ROWS_PER_W // CHUNK
NITER = NCHUNK // 2

IDX_BLK_CHUNKS = 20
IDXBLK = IDX_BLK_CHUNKS * CHUNK


def _sc_embed(day_i, time_i, locx_i, locy_i, day_table, time_table,
              locx_table, locy_table):
  mesh = plsc.VectorSubcoreMesh(core_axis_name="c", subcore_axis_name="s")

  @functools.partial(
      pl.kernel,
      mesh=mesh,
      compiler_params=pltpu.CompilerParams(use_tc_tiling_on_sc=False),
      out_type=jax.ShapeDtypeStruct((BL, D_OUT), jnp.float32),
      scratch_types=[
          pltpu.VMEM((2, IDXBLK), jnp.int32),
          pltpu.VMEM((2, IDXBLK), jnp.int32),
          pltpu.VMEM((2, IDXBLK), jnp.int32),
          pltpu.VMEM((2, IDXBLK), jnp.int32),
          pltpu.VMEM((2, CHUNK, D_DAY), jnp.float32),
          pltpu.VMEM((2, CHUNK, D_TIME), jnp.float32),
          pltpu.VMEM((2, CHUNK, D_LOC), jnp.float32),
          pltpu.VMEM((2, CHUNK, D_LOC), jnp.float32),
          pltpu.SemaphoreType.DMA,
          pltpu.SemaphoreType.DMA,
          pltpu.SemaphoreType.DMA,
          pltpu.SemaphoreType.DMA,
      ],
  )
  def k(day_h, time_h, locx_h, locy_h, dt_h, tt_h, xt_h, yt_h, out_h,
        di_v, ti_v, xi_v, yi_v, dr_v, tr_v, xr_v, yr_v, g0, g1, w0, w1):
    wid = lax.axis_index("s") * NUM_CORES + lax.axis_index("c")
    w_base = wid * ROWS_PER_W
    gsems = (g0, g1)
    wsems = (w0, w1)

    def load_idx_block(g):
      base = w_base + g * IDXBLK
      p = lax.rem(g, 2)
      pltpu.sync_copy(day_h.at[pl.ds(base, IDXBLK)], di_v.at[p])
      pltpu.sync_copy(time_h.at[pl.ds(base, IDXBLK)], ti_v.at[p])
      pltpu.sync_copy(locx_h.at[pl.ds(base, IDXBLK)], xi_v.at[p])
      pltpu.sync_copy(locy_h.at[pl.ds(base, IDXBLK)], yi_v.at[p])

    def gather_copies(c, b):
      p = lax.rem(c // IDX_BLK_CHUNKS, 2)
      off = lax.rem(c, IDX_BLK_CHUNKS) * CHUNK
      sem = gsems[b]
      return (
          pltpu.make_async_copy(
              dt_h.at[di_v.at[p, pl.ds(off, CHUNK)]], dr_v.at[b], sem),
          pltpu.make_async_copy(
              tt_h.at[ti_v.at[p, pl.ds(off, CHUNK)]], tr_v.at[b], sem),
          pltpu.make_async_copy(
              xt_h.at[xi_v.at[p, pl.ds(off, CHUNK)]], xr_v.at[b], sem),
          pltpu.make_async_copy(
              yt_h.at[yi_v.at[p, pl.ds(off, CHUNK)]], yr_v.at[b], sem),
      )

    def write_copies(c, b):
      base = w_base + c * CHUNK
      sem = wsems[b]
      rows = pl.ds(base, CHUNK)
      return (
          pltpu.make_async_copy(dr_v.at[b], out_h.at[rows, pl.ds(0, D_DAY)],
                                sem),
          pltpu.make_async_copy(tr_v.at[b],
                                out_h.at[rows, pl.ds(D_DAY, D_TIME)], sem),
          pltpu.make_async_copy(
              xr_v.at[b], out_h.at[rows, pl.ds(D_DAY + D_TIME, D_LOC)], sem),
          pltpu.make_async_copy(
              yr_v.at[b], out_h.at[rows,
                                   pl.ds(D_DAY + D_TIME + D_LOC, D_LOC)],
              sem),
      )

    def start_all(copies):
      for cp in copies:
        cp.start()

    def wait_all(copies):
      for cp in copies:
        cp.wait()

    load_idx_block(0)
    start_all(gather_copies(0, 0))
    start_all(gather_copies(1, 1))

    def body(i, carry):
      c0 = 2 * i
      c1 = c0 + 1
      wait_all(gather_copies(c0, 0))
      start_all(write_copies(c0, 0))
      wait_all(gather_copies(c1, 1))
      start_all(write_copies(c1, 1))

      @pl.when(i < NITER - 1)
      def _():
        @pl.when(lax.rem(c0 + 2, IDX_BLK_CHUNKS) == 0)
        def _():
          load_idx_block((c0 + 2) // IDX_BLK_CHUNKS)

        wait_all(write_copies(c0, 0))
        start_all(gather_copies(c0 + 2, 0))
        wait_all(write_copies(c1, 1))
        start_all(gather_copies(c1 + 2, 1))

      return carry

    lax.fori_loop(0, NITER, body, 0)

    wait_all(write_copies(NCHUNK - 2, 0))
    wait_all(write_copies(NCHUNK - 1, 1))

  return k(day_i, time_i, locx_i, locy_i, day_table, time_table, locx_table,
           locy_table)


def kernel(day, time, location_x, location_y, day_table, time_table,
           locx_table, locy_table):
  day_i = day.reshape(BL).astype(jnp.int32)
  time_i = time.reshape(BL).astype(jnp.int32)
  locx_i = location_x.reshape(BL).astype(jnp.int32)
  locy_i = location_y.reshape(BL).astype(jnp.int32)
  out = _sc_embed(day_i, time_i, locx_i, locy_i, day_table, time_table,
                  locx_table, locy_table)
  return out.reshape(B, L, D_OUT)


# speedup vs baseline: 1.8545x; 1.0021x over previous
import functools

import jax
import jax.numpy as jnp
from jax import lax
from jax.experimental import pallas as pl
from jax.experimental.pallas import tpu as pltpu
from jax.experimental.pallas import tpu_sc as plsc

B, L = 16384, 50
BL = B * L
D_DAY, D_TIME, D_LOC = 32, 32, 128
D_OUT = D_DAY + D_TIME + 2 * D_LOC

NUM_CORES = 2
NUM_SUBCORES = 16
NW = NUM_CORES * NUM_SUBCORES
ROWS_PER_W = BL // NW
CHUNK = 128
NCHUNK = ROWS_PER_W // CHUNK

IDX_BLK_CHUNKS = 10
IDXBLK = IDX_BLK_CHUNKS * CHUNK


def _sc_embed(day_i, time_i, locx_i, locy_i, day_table, time_table,
              locx_table, locy_table):
  mesh = plsc.VectorSubcoreMesh(core_axis_name="c", subcore_axis_name="s")

  @functools.partial(
      pl.kernel,
      mesh=mesh,
      compiler_params=pltpu.CompilerParams(use_tc_tiling_on_sc=False),
      out_type=jax.ShapeDtypeStruct((BL, D_OUT), jnp.float32),
      scratch_types=[
          pltpu.VMEM((IDXBLK,), jnp.int32),
          pltpu.VMEM((IDXBLK,), jnp.int32),
          pltpu.VMEM((IDXBLK,), jnp.int32),
          pltpu.VMEM((IDXBLK,), jnp.int32),
          pltpu.VMEM((CHUNK, D_DAY), jnp.float32),
          pltpu.VMEM((CHUNK, D_TIME), jnp.float32),
          pltpu.VMEM((CHUNK, D_LOC), jnp.float32),
          pltpu.VMEM((CHUNK, D_LOC), jnp.float32),
          pltpu.VMEM((2, CHUNK, D_OUT), jnp.float32),
          pltpu.SemaphoreType.DMA,
          pltpu.SemaphoreType.DMA,
          pltpu.SemaphoreType.DMA,
      ],
  )
  def k(day_h, time_h, locx_h, locy_h, dt_h, tt_h, xt_h, yt_h, out_h,
        di_v, ti_v, xi_v, yi_v, dr_v, tr_v, xr_v, yr_v, asm_v,
        gsem, w0, w1):
    wid = lax.axis_index("s") * NUM_CORES + lax.axis_index("c")
    w_base = wid * ROWS_PER_W
    wsems = (w0, w1)

    def load_idx_block(g):
      base = w_base + g * IDXBLK
      pltpu.sync_copy(day_h.at[pl.ds(base, IDXBLK)], di_v)
      pltpu.sync_copy(time_h.at[pl.ds(base, IDXBLK)], ti_v)
      pltpu.sync_copy(locx_h.at[pl.ds(base, IDXBLK)], xi_v)
      pltpu.sync_copy(locy_h.at[pl.ds(base, IDXBLK)], yi_v)

    def gather_copies(c):
      off = lax.rem(c, IDX_BLK_CHUNKS) * CHUNK
      return (
          pltpu.make_async_copy(
              dt_h.at[di_v.at[pl.ds(off, CHUNK)]], dr_v, gsem),
          pltpu.make_async_copy(
              tt_h.at[ti_v.at[pl.ds(off, CHUNK)]], tr_v, gsem),
          pltpu.make_async_copy(
              xt_h.at[xi_v.at[pl.ds(off, CHUNK)]], xr_v, gsem),
          pltpu.make_async_copy(
              yt_h.at[yi_v.at[pl.ds(off, CHUNK)]], yr_v, gsem),
      )

    def write_copy(c, a):
      base = w_base + c * CHUNK
      return pltpu.make_async_copy(asm_v.at[a], out_h.at[pl.ds(base, CHUNK)],
                                   wsems[a])

    def assemble(a):
      def row_body(r, carry):
        for j in range(D_DAY // 16):
          asm_v[a, r, pl.ds(16 * j, 16)] = dr_v[r, pl.ds(16 * j, 16)]
        for j in range(D_TIME // 16):
          asm_v[a, r, pl.ds(D_DAY + 16 * j, 16)] = tr_v[r, pl.ds(16 * j, 16)]
        for j in range(D_LOC // 16):
          asm_v[a, r, pl.ds(D_DAY + D_TIME + 16 * j, 16)] = (
              xr_v[r, pl.ds(16 * j, 16)])
        for j in range(D_LOC // 16):
          asm_v[a, r, pl.ds(D_DAY + D_TIME + D_LOC + 16 * j, 16)] = (
              yr_v[r, pl.ds(16 * j, 16)])
        return carry

      lax.fori_loop(0, CHUNK, row_body, 0)

    load_idx_block(0)
    for cp in gather_copies(0):
      cp.start()

    def step(c, a):
      @pl.when(c >= 2)
      def _():
        write_copy(c - 2, a).wait()

      for cp in gather_copies(c):
        cp.wait()
      assemble(a)

      @pl.when(c + 1 < NCHUNK)
      def _():
        @pl.when(lax.rem(c + 1, IDX_BLK_CHUNKS) == 0)
        def _():
          load_idx_block((c + 1) // IDX_BLK_CHUNKS)

        for cp in gather_copies(c + 1):
          cp.start()

      write_copy(c, a).start()

    def body(i, carry):
      c0 = 2 * i
      step(c0, 0)
      step(c0 + 1, 1)
      return carry

    lax.fori_loop(0, NCHUNK // 2, body, 0)

    write_copy(NCHUNK - 2, 0).wait()
    write_copy(NCHUNK - 1, 1).wait()

  return k(day_i, time_i, locx_i, locy_i, day_table, time_table, locx_table,
           locy_table)


def kernel(day, time, location_x, location_y, day_table, time_table,
           locx_table, locy_table):
  day_i = day.reshape(BL).astype(jnp.int32)
  time_i = time.reshape(BL).astype(jnp.int32)
  locx_i = location_x.reshape(BL).astype(jnp.int32)
  locy_i = location_y.reshape(BL).astype(jnp.int32)
  out = _sc_embed(day_i, time_i, locx_i, locy_i, day_table, time_table,
                  locx_table, locy_table)
  return out.reshape(B, L, D_OUT)


# speedup vs baseline: 3.9232x; 2.1155x over previous
import functools

import jax
import jax.numpy as jnp
from jax import lax
from jax.experimental import pallas as pl
from jax.experimental.pallas import tpu as pltpu
from jax.experimental.pallas import tpu_sc as plsc

B, L = 16384, 50
BL = B * L
D_DAY, D_TIME, D_LOC = 32, 32, 128
D_OUT = D_DAY + D_TIME + 2 * D_LOC
N_DAY, N_TIME, N_LOC = 7, 48, 201

NUM_CORES = 2
NUM_SUBCORES = 16
NW = NUM_CORES * NUM_SUBCORES
ROWS_PER_W = BL // NW
CHUNK = 80
NCHUNK = ROWS_PER_W // CHUNK

IDX_BLK_CHUNKS = 10
IDXBLK = IDX_BLK_CHUNKS * CHUNK


def _sc_embed(day_i, time_i, locx_i, locy_i, day_table, time_table,
              locx_table, locy_table):
  mesh = plsc.VectorSubcoreMesh(core_axis_name="c", subcore_axis_name="s")

  @functools.partial(
      pl.kernel,
      mesh=mesh,
      compiler_params=pltpu.CompilerParams(use_tc_tiling_on_sc=False),
      out_type=jax.ShapeDtypeStruct((BL, D_OUT), jnp.float32),
      scratch_types=[
          pltpu.VMEM((IDXBLK,), jnp.int32),
          pltpu.VMEM((IDXBLK,), jnp.int32),
          pltpu.VMEM((IDXBLK,), jnp.int32),
          pltpu.VMEM((IDXBLK,), jnp.int32),
          pltpu.VMEM((N_DAY * D_DAY,), jnp.float32),
          pltpu.VMEM((N_TIME * D_TIME,), jnp.float32),
          pltpu.VMEM((N_LOC * D_LOC,), jnp.float32),
          pltpu.VMEM((N_LOC * D_LOC,), jnp.float32),
          pltpu.VMEM((2, CHUNK, D_OUT), jnp.float32),
          pltpu.SemaphoreType.DMA,
          pltpu.SemaphoreType.DMA,
      ],
  )
  def k(day_h, time_h, locx_h, locy_h, dt_h, tt_h, xt_h, yt_h, out_h,
        di_v, ti_v, xi_v, yi_v, dt_v, tt_v, xt_v, yt_v, asm_v, w0, w1):
    wid = lax.axis_index("s") * NUM_CORES + lax.axis_index("c")
    w_base = wid * ROWS_PER_W
    wsems = (w0, w1)

    pltpu.sync_copy(dt_h, dt_v)
    pltpu.sync_copy(tt_h, tt_v)
    pltpu.sync_copy(xt_h, xt_v)
    pltpu.sync_copy(yt_h, yt_v)

    def load_idx_block(g):
      base = w_base + g * IDXBLK
      pltpu.sync_copy(day_h.at[pl.ds(base, IDXBLK)], di_v)
      pltpu.sync_copy(time_h.at[pl.ds(base, IDXBLK)], ti_v)
      pltpu.sync_copy(locx_h.at[pl.ds(base, IDXBLK)], xi_v)
      pltpu.sync_copy(locy_h.at[pl.ds(base, IDXBLK)], yi_v)

    def write_copy(c, a):
      base = w_base + c * CHUNK
      return pltpu.make_async_copy(asm_v.at[a], out_h.at[pl.ds(base, CHUNK)],
                                   wsems[a])

    def assemble(c, a):
      off = lax.rem(c, IDX_BLK_CHUNKS) * CHUNK

      def grp_body(g, carry):
        r0 = g * 16
        dv = di_v[pl.ds(off + r0, 16)] * D_DAY
        tv = ti_v[pl.ds(off + r0, 16)] * D_TIME
        xv = xi_v[pl.ds(off + r0, 16)] * D_LOC
        yv = yi_v[pl.ds(off + r0, 16)] * D_LOC
        for rr in range(16):
          r = r0 + rr
          s_d = dv[rr]
          s_t = tv[rr]
          s_x = xv[rr]
          s_y = yv[rr]
          for j in range(D_DAY // 16):
            asm_v[a, r, pl.ds(16 * j, 16)] = dt_v[pl.ds(s_d + 16 * j, 16)]
          for j in range(D_TIME // 16):
            asm_v[a, r, pl.ds(D_DAY + 16 * j, 16)] = (
                tt_v[pl.ds(s_t + 16 * j, 16)])
          for j in range(D_LOC // 16):
            asm_v[a, r, pl.ds(D_DAY + D_TIME + 16 * j, 16)] = (
                xt_v[pl.ds(s_x + 16 * j, 16)])
          for j in range(D_LOC // 16):
            asm_v[a, r, pl.ds(D_DAY + D_TIME + D_LOC + 16 * j, 16)] = (
                yt_v[pl.ds(s_y + 16 * j, 16)])
        return carry

      lax.fori_loop(0, CHUNK // 16, grp_body, 0)

    load_idx_block(0)

    def step(c, a):
      @pl.when(lax.rem(c, IDX_BLK_CHUNKS) == 0)
      def _():
        @pl.when(c > 0)
        def _():
          load_idx_block(c // IDX_BLK_CHUNKS)

      @pl.when(c >= 2)
      def _():
        write_copy(c - 2, a).wait()

      assemble(c, a)
      write_copy(c, a).start()

    def body(i, carry):
      c0 = 2 * i
      step(c0, 0)
      step(c0 + 1, 1)
      return carry

    lax.fori_loop(0, NCHUNK // 2, body, 0)

    write_copy(NCHUNK - 2, 0).wait()
    write_copy(NCHUNK - 1, 1).wait()

  return k(day_i, time_i, locx_i, locy_i,
           day_table.reshape(N_DAY * D_DAY),
           time_table.reshape(N_TIME * D_TIME),
           locx_table.reshape(N_LOC * D_LOC),
           locy_table.reshape(N_LOC * D_LOC))


def kernel(day, time, location_x, location_y, day_table, time_table,
           locx_table, locy_table):
  day_i = day.reshape(BL).astype(jnp.int32)
  time_i = time.reshape(BL).astype(jnp.int32)
  locx_i = location_x.reshape(BL).astype(jnp.int32)
  locy_i = location_y.reshape(BL).astype(jnp.int32)
  out = _sc_embed(day_i, time_i, locx_i, locy_i, day_table, time_table,
                  locx_table, locy_table)
  return out.reshape(B, L, D_OUT)


# speedup vs baseline: 4.1661x; 1.0619x over previous
import functools

import jax
import jax.numpy as jnp
from jax import lax
from jax.experimental import pallas as pl
from jax.experimental.pallas import tpu as pltpu
from jax.experimental.pallas import tpu_sc as plsc

B, L = 16384, 50
BL = B * L
D_DAY, D_TIME, D_LOC = 32, 32, 128
D_OUT = D_DAY + D_TIME + 2 * D_LOC
N_DAY, N_TIME, N_LOC = 7, 48, 201

NUM_CORES = 2
NUM_SUBCORES = 16
NW = NUM_CORES * NUM_SUBCORES
ELEMS_PER_W = B // NW
ROWS_PER_W = ELEMS_PER_W * L

IDX_BLK_ELEMS = 20
IDXBLK = IDX_BLK_ELEMS * L


def _sc_embed(day_i, time_i, locx_i, locy_i, day_table, time_table,
              locx_table, locy_table):
  mesh = plsc.VectorSubcoreMesh(core_axis_name="c", subcore_axis_name="s")

  @functools.partial(
      pl.kernel,
      mesh=mesh,
      out_type=jax.ShapeDtypeStruct((B, L, D_OUT), jnp.float32),
      scratch_types=[
          pltpu.VMEM((IDXBLK,), jnp.int32),
          pltpu.VMEM((IDXBLK,), jnp.int32),
          pltpu.VMEM((IDXBLK,), jnp.int32),
          pltpu.VMEM((IDXBLK,), jnp.int32),
          pltpu.VMEM((N_DAY * D_DAY,), jnp.float32),
          pltpu.VMEM((N_TIME * D_TIME,), jnp.float32),
          pltpu.VMEM((N_LOC * D_LOC,), jnp.float32),
          pltpu.VMEM((N_LOC * D_LOC,), jnp.float32),
          pltpu.VMEM((2, L, D_OUT), jnp.float32),
          pltpu.SemaphoreType.DMA,
          pltpu.SemaphoreType.DMA,
      ],
  )
  def k(day_h, time_h, locx_h, locy_h, dt_h, tt_h, xt_h, yt_h, out_h,
        di_v, ti_v, xi_v, yi_v, dt_v, tt_v, xt_v, yt_v, asm_v, w0, w1):
    wid = lax.axis_index("s") * NUM_CORES + lax.axis_index("c")
    e_base = wid * ELEMS_PER_W
    wsems = (w0, w1)

    pltpu.sync_copy(dt_h, dt_v)
    pltpu.sync_copy(tt_h, tt_v)
    pltpu.sync_copy(xt_h, xt_v)
    pltpu.sync_copy(yt_h, yt_v)

    def load_idx_block(g):
      base = e_base * L + g * IDXBLK
      pltpu.sync_copy(day_h.at[pl.ds(base, IDXBLK)], di_v)
      pltpu.sync_copy(time_h.at[pl.ds(base, IDXBLK)], ti_v)
      pltpu.sync_copy(locx_h.at[pl.ds(base, IDXBLK)], xi_v)
      pltpu.sync_copy(locy_h.at[pl.ds(base, IDXBLK)], yi_v)

    def write_copy(e, a):
      return pltpu.make_async_copy(asm_v.at[a], out_h.at[e_base + e],
                                   wsems[a])

    def assemble(e, a):
      off = lax.rem(e, IDX_BLK_ELEMS) * L

      def rows(r0, n, idx_off):
        dv = di_v[pl.ds(off + idx_off, 16)] * D_DAY
        tv = ti_v[pl.ds(off + idx_off, 16)] * D_TIME
        xv = xi_v[pl.ds(off + idx_off, 16)] * D_LOC
        yv = yi_v[pl.ds(off + idx_off, 16)] * D_LOC
        for rr in range(n):
          r = r0 + rr
          lane = r - idx_off
          s_d = dv[lane]
          s_t = tv[lane]
          s_x = xv[lane]
          s_y = yv[lane]
          for j in range(D_DAY // 16):
            asm_v[a, r, pl.ds(16 * j, 16)] = dt_v[pl.ds(s_d + 16 * j, 16)]
          for j in range(D_TIME // 16):
            asm_v[a, r, pl.ds(D_DAY + 16 * j, 16)] = (
                tt_v[pl.ds(s_t + 16 * j, 16)])
          for j in range(D_LOC // 16):
            asm_v[a, r, pl.ds(D_DAY + D_TIME + 16 * j, 16)] = (
                xt_v[pl.ds(s_x + 16 * j, 16)])
          for j in range(D_LOC // 16):
            asm_v[a, r, pl.ds(D_DAY + D_TIME + D_LOC + 16 * j, 16)] = (
                yt_v[pl.ds(s_y + 16 * j, 16)])

      rows(0, 16, 0)
      rows(16, 16, 16)
      rows(32, 16, 32)
      rows(48, 2, 34)

    load_idx_block(0)

    def step(e, a):
      @pl.when(lax.rem(e, IDX_BLK_ELEMS) == 0)
      def _():
        @pl.when(e > 0)
        def _():
          load_idx_block(e // IDX_BLK_ELEMS)

      @pl.when(e >= 2)
      def _():
        write_copy(e - 2, a).wait()

      assemble(e, a)
      write_copy(e, a).start()

    def body(i, carry):
      e0 = 2 * i
      step(e0, 0)
      step(e0 + 1, 1)
      return carry

    lax.fori_loop(0, ELEMS_PER_W // 2, body, 0)

    write_copy(ELEMS_PER_W - 2, 0).wait()
    write_copy(ELEMS_PER_W - 1, 1).wait()

  return k(day_i, time_i, locx_i, locy_i,
           day_table.reshape(N_DAY * D_DAY),
           time_table.reshape(N_TIME * D_TIME),
           locx_table.reshape(N_LOC * D_LOC),
           locy_table.reshape(N_LOC * D_LOC))


def kernel(day, time, location_x, location_y, day_table, time_table,
           locx_table, locy_table):
  day_i = day.reshape(BL).astype(jnp.int32)
  time_i = time.reshape(BL).astype(jnp.int32)
  locx_i = location_x.reshape(BL).astype(jnp.int32)
  locy_i = location_y.reshape(BL).astype(jnp.int32)
  return _sc_embed(day_i, time_i, locx_i, locy_i, day_table, time_table,
                   locx_table, locy_table)
